# baseline (speedup 1.0000x reference)
import functools

import jax
import jax.numpy as jnp
from jax import lax
from jax.experimental import pallas as pl
from jax.experimental.pallas import tpu as pltpu
from jax.experimental.pallas import tpu_sc as plsc

N_NODES = 10000
DEGREE = 32
D = 128
DH = D // 2

NW = 32
CHUNK = 144
NBUF = 2

PARTS = (4000, 4000, 2000)

BLK = 1000
KP = DEGREE // 2


def _pad_rows(n):
  q = NW * CHUNK * NBUF
  return -(-n // q) * q


@functools.cache
def _make_sc_gather(total_rows):
  rows_per_w = total_rows // NW
  nchunk = rows_per_w // CHUNK
  mesh = plsc.VectorSubcoreMesh(core_axis_name="c", subcore_axis_name="s")

  @functools.partial(
      pl.kernel,
      mesh=mesh,
      out_type=jax.ShapeDtypeStruct((total_rows, D), jnp.float32),
      scratch_types=(
          [pltpu.VMEM((CHUNK,), jnp.int32)] * NBUF
          + [pltpu.VMEM((CHUNK, D), jnp.float32)] * NBUF
          + [pltpu.SemaphoreType.DMA] * (2 * NBUF)
      ),
  )
  def sc_gather(table, idx, out, *bufs):
    idxs = bufs[:NBUF]
    rows = bufs[NBUF:2 * NBUF]
    gsem = bufs[2 * NBUF:3 * NBUF]
    wsem = bufs[3 * NBUF:4 * NBUF]
    nc = 2
    wid = lax.axis_index("s") * nc + lax.axis_index("c")
    base = wid * rows_per_w
    rounds = nchunk // NBUF

    def ioff(c):
      return pl.multiple_of(base + c * CHUNK, 8)

    for j in range(NBUF):
      pltpu.sync_copy(idx.at[pl.ds(ioff(j), CHUNK)], idxs[j])
      pltpu.async_copy(table.at[idxs[j]], rows[j], gsem[j])

    def body(r, carry):
      for j in range(NBUF):
        c = NBUF * r + j
        pltpu.make_async_copy(table.at[idxs[j]], rows[j], gsem[j]).wait()
        pltpu.async_copy(rows[j], out.at[pl.ds(ioff(c), CHUNK)], wsem[j])
        pltpu.make_async_copy(
            rows[j], out.at[pl.ds(ioff(c), CHUNK)], wsem[j]).wait()

        @pl.when(r < rounds - 1)
        def _(j=j, c=c):
          pltpu.sync_copy(idx.at[pl.ds(ioff(c + NBUF), CHUNK)], idxs[j])
          pltpu.async_copy(table.at[idxs[j]], rows[j], gsem[j])
      return carry

    lax.fori_loop(0, rounds, body, 0)

  return sc_gather


def _tc_body(ea_ref, eb_ref, ur_ref, w1d_ref, w1b_ref, w2d_ref, w3_ref,
             b1_ref, b2_ref, o_ref, s_ref, acc_ref, l_ref):
  k = pl.program_id(1)
  ea = ea_ref[...]
  eb = eb_ref[...]

  @pl.when(k == 0)
  def _():
    xu = ur_ref[...].astype(jnp.bfloat16)
    s_ref[...] = jnp.dot(xu, w1b_ref[...], preferred_element_type=jnp.float32)

  s = s_ref[...]
  x2 = jnp.concatenate([ea, eb], axis=1).astype(jnp.bfloat16)
  h1 = jnp.dot(x2, w1d_ref[...], preferred_element_type=jnp.float32)
  s2 = jnp.concatenate([s, s], axis=1)
  h1 = jnp.maximum(h1 + s2 + b1_ref[...], 0.0).astype(jnp.bfloat16)
  h2 = jnp.dot(h1, w2d_ref[...], preferred_element_type=jnp.float32)
  h2 = jnp.maximum(h2 + b2_ref[...], 0.0)
  w3c = w3_ref[...]
  la = jnp.sum(h2[:, :D] * w3c, axis=1, keepdims=True)
  lb = jnp.sum(h2[:, D:] * w3c, axis=1, keepdims=True)
  wa = jnp.exp(la)
  wb = jnp.exp(lb)

  @pl.when(k == 0)
  def _():
    l_ref[...] = wa + wb
    acc_ref[...] = wa * ea + wb * eb

  @pl.when(k > 0)
  def _():
    l_ref[...] = l_ref[...] + wa + wb
    acc_ref[...] = acc_ref[...] + wa * ea + wb * eb

  @pl.when(k == pl.num_programs(1) - 1)
  def _():
    o_ref[...] = acc_ref[...] / l_ref[...]


@functools.cache
def _make_tc_attend(nh):
  nb = nh // BLK
  e_blocks = nh * DEGREE // BLK
  return pl.pallas_call(
      _tc_body,
      grid=(nb, KP),
      in_specs=[
          pl.BlockSpec((BLK, D), lambda b, k: (2 * k * nb + b, 0)),
          pl.BlockSpec((BLK, D), lambda b, k: ((2 * k + 1) * nb + b, 0)),
          pl.BlockSpec((BLK, D), lambda b, k: (e_blocks + b, 0)),
          pl.BlockSpec((2 * D, 2 * D), lambda b, k: (0, 0)),
          pl.BlockSpec((D, D), lambda b, k: (0, 0)),
          pl.BlockSpec((2 * D, 2 * D), lambda b, k: (0, 0)),
          pl.BlockSpec((1, D), lambda b, k: (0, 0)),
          pl.BlockSpec((1, 2 * D), lambda b, k: (0, 0)),
          pl.BlockSpec((1, 2 * D), lambda b, k: (0, 0)),
      ],
      out_specs=pl.BlockSpec((BLK, D), lambda b, k: (b, 0)),
      out_shape=jax.ShapeDtypeStruct((nh, D), jnp.float32),
      scratch_shapes=[
          pltpu.VMEM((BLK, D), jnp.float32),
          pltpu.VMEM((BLK, D), jnp.float32),
          pltpu.VMEM((BLK, 1), jnp.float32),
      ],
  )


def _blkdiag(w):
  z = jnp.zeros_like(w)
  return jnp.concatenate(
      [jnp.concatenate([w, z], axis=1), jnp.concatenate([z, w], axis=1)],
      axis=0)


def kernel(nodes, to_neighs, u2e, W1, b1, W2, b2, W3, b3):
  w1d = _blkdiag(W1[:D]).astype(jnp.bfloat16)
  w2d = _blkdiag(W2).astype(jnp.bfloat16)
  w1b = W1[D:].astype(jnp.bfloat16)
  w3t = W3.reshape(1, D)
  b1t = jnp.tile(b1.reshape(1, D), (1, 2))
  b2t = jnp.tile(b2.reshape(1, D), (1, 2))
  nodes = nodes.astype(jnp.int32)
  to_neighs = to_neighs.astype(jnp.int32)
  outs = []
  lo = 0
  for nh in PARTS:
    hi = lo + nh
    e_rows = nh * DEGREE
    total_rows = _pad_rows(e_rows + nh)
    pad = total_rows - e_rows - nh
    idx_i = jnp.concatenate(
        [to_neighs[lo:hi].T.reshape(-1), nodes[lo:hi],
         jnp.zeros((pad,), jnp.int32)])
    g = _make_sc_gather(total_rows)(u2e, idx_i)
    outs.append(_make_tc_attend(nh)(g, g, g, w1d, w1b, w2d, w3t, b1t, b2t))
    lo = hi
  return jnp.concatenate(outs, axis=0)


# speedup vs baseline: 3.2402x; 3.2402x over previous
import functools

import jax
import jax.numpy as jnp
from jax import lax
from jax.experimental import pallas as pl
from jax.experimental.pallas import tpu as pltpu
from jax.experimental.pallas import tpu_sc as plsc

N_NODES = 10000
DEGREE = 32
D = 128
DH = D // 2

NW = 32
CHUNK = 144
NBUF = 4

PARTS = (5000, 5000)

BLK = 1000
KP = DEGREE // 2


def _pad_rows(n):
  q = NW * CHUNK * NBUF
  return -(-n // q) * q


@functools.cache
def _make_sc_gather(total_rows):
  rows_per_w = total_rows // NW
  nchunk = rows_per_w // CHUNK
  mesh = plsc.VectorSubcoreMesh(core_axis_name="c", subcore_axis_name="s")

  @functools.partial(
      pl.kernel,
      mesh=mesh,
      out_type=jax.ShapeDtypeStruct((total_rows, D), jnp.float32),
      scratch_types=(
          [pltpu.VMEM((CHUNK,), jnp.int32)] * NBUF
          + [pltpu.VMEM((CHUNK, D), jnp.float32)] * NBUF
          + [pltpu.SemaphoreType.DMA] * (2 * NBUF)
      ),
  )
  def sc_gather(table, idx, out, *bufs):
    idxs = bufs[:NBUF]
    rows = bufs[NBUF:2 * NBUF]
    gsem = bufs[2 * NBUF:3 * NBUF]
    wsem = bufs[3 * NBUF:4 * NBUF]
    nc = 2
    wid = lax.axis_index("s") * nc + lax.axis_index("c")
    base = wid * rows_per_w
    rounds = nchunk // NBUF

    def ioff(c):
      return pl.multiple_of(base + c * CHUNK, 8)

    for j in range(NBUF):
      pltpu.sync_copy(idx.at[pl.ds(ioff(j), CHUNK)], idxs[j])
      pltpu.async_copy(table.at[idxs[j]], rows[j], gsem[j])

    def body(r, carry):
      for j in range(NBUF):
        c = NBUF * r + j
        pltpu.make_async_copy(table.at[idxs[j]], rows[j], gsem[j]).wait()
        pltpu.async_copy(rows[j], out.at[pl.ds(ioff(c), CHUNK)], wsem[j])
        pltpu.make_async_copy(
            rows[j], out.at[pl.ds(ioff(c), CHUNK)], wsem[j]).wait()

        @pl.when(r < rounds - 1)
        def _(j=j, c=c):
          pltpu.sync_copy(idx.at[pl.ds(ioff(c + NBUF), CHUNK)], idxs[j])
          pltpu.async_copy(table.at[idxs[j]], rows[j], gsem[j])
      return carry

    lax.fori_loop(0, rounds, body, 0)

  return sc_gather


def _tc_body(ea_ref, eb_ref, ur_ref, w1d_ref, w1b_ref, w2d_ref, w3_ref,
             b1_ref, b2_ref, o_ref, s_ref, acc_ref, l_ref):
  k = pl.program_id(1)
  ea = ea_ref[...]
  eb = eb_ref[...]

  @pl.when(k == 0)
  def _():
    xu = ur_ref[...].astype(jnp.bfloat16)
    s_ref[...] = jnp.dot(xu, w1b_ref[...], preferred_element_type=jnp.float32)

  s = s_ref[...]
  x2 = jnp.concatenate([ea, eb], axis=1).astype(jnp.bfloat16)
  h1 = jnp.dot(x2, w1d_ref[...], preferred_element_type=jnp.float32)
  s2 = jnp.concatenate([s, s], axis=1)
  h1 = jnp.maximum(h1 + s2 + b1_ref[...], 0.0).astype(jnp.bfloat16)
  h2 = jnp.dot(h1, w2d_ref[...], preferred_element_type=jnp.float32)
  h2 = jnp.maximum(h2 + b2_ref[...], 0.0)
  w3c = w3_ref[...]
  la = jnp.sum(h2[:, :D] * w3c, axis=1, keepdims=True)
  lb = jnp.sum(h2[:, D:] * w3c, axis=1, keepdims=True)
  wa = jnp.exp(la)
  wb = jnp.exp(lb)

  @pl.when(k == 0)
  def _():
    l_ref[...] = wa + wb
    acc_ref[...] = wa * ea + wb * eb

  @pl.when(k > 0)
  def _():
    l_ref[...] = l_ref[...] + wa + wb
    acc_ref[...] = acc_ref[...] + wa * ea + wb * eb

  @pl.when(k == pl.num_programs(1) - 1)
  def _():
    o_ref[...] = acc_ref[...] / l_ref[...]


@functools.cache
def _make_tc_attend(nh):
  nb = nh // BLK
  e_blocks = nh * DEGREE // BLK
  return pl.pallas_call(
      _tc_body,
      grid=(nb, KP),
      in_specs=[
          pl.BlockSpec((BLK, D), lambda b, k: (2 * k * nb + b, 0)),
          pl.BlockSpec((BLK, D), lambda b, k: ((2 * k + 1) * nb + b, 0)),
          pl.BlockSpec((BLK, D), lambda b, k: (e_blocks + b, 0)),
          pl.BlockSpec((2 * D, 2 * D), lambda b, k: (0, 0)),
          pl.BlockSpec((D, D), lambda b, k: (0, 0)),
          pl.BlockSpec((2 * D, 2 * D), lambda b, k: (0, 0)),
          pl.BlockSpec((1, D), lambda b, k: (0, 0)),
          pl.BlockSpec((1, 2 * D), lambda b, k: (0, 0)),
          pl.BlockSpec((1, 2 * D), lambda b, k: (0, 0)),
      ],
      out_specs=pl.BlockSpec((BLK, D), lambda b, k: (b, 0)),
      out_shape=jax.ShapeDtypeStruct((nh, D), jnp.float32),
      scratch_shapes=[
          pltpu.VMEM((BLK, D), jnp.float32),
          pltpu.VMEM((BLK, D), jnp.float32),
          pltpu.VMEM((BLK, 1), jnp.float32),
      ],
  )


def _blkdiag(w):
  z = jnp.zeros_like(w)
  return jnp.concatenate(
      [jnp.concatenate([w, z], axis=1), jnp.concatenate([z, w], axis=1)],
      axis=0)


def kernel(nodes, to_neighs, u2e, W1, b1, W2, b2, W3, b3):
  w1d = _blkdiag(W1[:D]).astype(jnp.bfloat16)
  w2d = _blkdiag(W2).astype(jnp.bfloat16)
  w1b = W1[D:].astype(jnp.bfloat16)
  w3t = W3.reshape(1, D)
  b1t = jnp.tile(b1.reshape(1, D), (1, 2))
  b2t = jnp.tile(b2.reshape(1, D), (1, 2))
  nodes = nodes.astype(jnp.int32)
  to_neighs = to_neighs.astype(jnp.int32)
  outs = []
  lo = 0
  for nh in PARTS:
    hi = lo + nh
    e_rows = nh * DEGREE
    total_rows = _pad_rows(e_rows + nh)
    pad = total_rows - e_rows - nh
    idx_i = jnp.concatenate(
        [to_neighs[lo:hi].T.reshape(-1), nodes[lo:hi],
         jnp.zeros((pad,), jnp.int32)])
    g = _make_sc_gather(total_rows)(u2e, idx_i)
    outs.append(_make_tc_attend(nh)(g, g, g, w1d, w1b, w2d, w3t, b1t, b2t))
    lo = hi
  return jnp.concatenate(outs, axis=0)
